# Initial kernel scaffold; baseline (speedup 1.0000x reference)
import jax
import jax.numpy as jnp
from jax.experimental import pallas as pl


def kernel(token_ids, weight):
    raise NotImplementedError("write your pallas kernel here")


# speedup vs baseline: 1.8000x; 1.8000x over previous
import functools

import jax
import jax.numpy as jnp
from jax import lax
from jax.experimental import pallas as pl
from jax.experimental.pallas import tpu as pltpu
from jax.experimental.pallas import tpu_sc as plsc

NC = 2
NS = 16
NW = NC * NS

CHUNK = 512


@functools.partial(jax.jit, static_argnames=("B", "D"))
def _embedding_gather(table, idx_flat, *, B, D):
    rows_per_w = B // NW
    n_chunks = rows_per_w // CHUNK
    mesh = plsc.VectorSubcoreMesh(core_axis_name="c", subcore_axis_name="s")

    @functools.partial(
        pl.kernel,
        mesh=mesh,
        out_type=jax.ShapeDtypeStruct((B, D), jnp.float32),
        scratch_types=[
            pltpu.VMEM((CHUNK,), jnp.int32),
            pltpu.VMEM((CHUNK, D), jnp.float32),
            pltpu.SemaphoreType.DMA,
        ],
        compiler_params=pltpu.CompilerParams(use_tc_tiling_on_sc=False),
    )
    def k(table_hbm, idx_hbm, out_hbm, idx_v, rows_v, sem):
        wid = lax.axis_index("s") * NC + lax.axis_index("c")
        base = wid * rows_per_w

        def body(c, carry):
            off = base + c * CHUNK
            pltpu.sync_copy(idx_hbm.at[pl.ds(off, CHUNK)], idx_v)
            pltpu.async_copy(table_hbm.at[idx_v], rows_v, sem).wait()
            pltpu.sync_copy(rows_v, out_hbm.at[pl.ds(off, CHUNK)])
            return carry

        lax.fori_loop(0, n_chunks, body, 0)

    return k(table, idx_flat)


def kernel(token_ids, weight):
    S, T = token_ids.shape
    D = weight.shape[1]
    B = S * T
    idx_flat = token_ids.reshape(B).astype(jnp.int32)
    out = _embedding_gather(weight, idx_flat, B=B, D=D)
    return out.reshape(S, T, D)


# speedup vs baseline: 1.8714x; 1.0397x over previous
import functools

import jax
import jax.numpy as jnp
from jax import lax
from jax.experimental import pallas as pl
from jax.experimental.pallas import tpu as pltpu
from jax.experimental.pallas import tpu_sc as plsc

NC = 2
NS = 16
NW = NC * NS

CHUNK = 512


@functools.partial(jax.jit, static_argnames=("B", "D"))
def _embedding_gather(table, idx_flat, *, B, D):
    rows_per_w = B // NW
    n_chunks = rows_per_w // CHUNK
    assert n_chunks % 2 == 0
    n_groups = n_chunks // 2
    mesh = plsc.VectorSubcoreMesh(core_axis_name="c", subcore_axis_name="s")

    @functools.partial(
        pl.kernel,
        mesh=mesh,
        out_type=jax.ShapeDtypeStruct((B, D), jnp.float32),
        scratch_types=[
            pltpu.VMEM((rows_per_w,), jnp.int32),
            pltpu.VMEM((CHUNK, D), jnp.float32),
            pltpu.VMEM((CHUNK, D), jnp.float32),
            pltpu.SemaphoreType.DMA,
            pltpu.SemaphoreType.DMA,
            pltpu.SemaphoreType.DMA,
            pltpu.SemaphoreType.DMA,
        ],
        compiler_params=pltpu.CompilerParams(use_tc_tiling_on_sc=False),
    )
    def k(table_hbm, idx_hbm, out_hbm, idx_v, buf_a, buf_b, gsem_a, gsem_b,
          wsem_a, wsem_b):
        wid = lax.axis_index("s") * NC + lax.axis_index("c")
        base = wid * rows_per_w
        pltpu.sync_copy(idx_hbm.at[pl.ds(base, rows_per_w)], idx_v)

        def gather(c, buf, sem):
            return pltpu.async_copy(
                table_hbm.at[idx_v.at[pl.ds(c * CHUNK, CHUNK)]], buf, sem)

        def writeback(c, buf, sem):
            return pltpu.async_copy(
                buf, out_hbm.at[pl.ds(base + c * CHUNK, CHUNK)], sem)

        def wait_fill(buf, sem):
            pltpu.make_async_copy(out_hbm.at[pl.ds(0, CHUNK)], buf, sem).wait()

        def wait_drain(buf, sem):
            pltpu.make_async_copy(buf, out_hbm.at[pl.ds(0, CHUNK)], sem).wait()

        gather(0, buf_a, gsem_a)

        def body(g, carry):
            c0 = 2 * g
            wait_fill(buf_a, gsem_a)
            writeback(c0, buf_a, wsem_a)
            @pl.when(g > 0)
            def _():
                wait_drain(buf_b, wsem_b)
            gather(c0 + 1, buf_b, gsem_b)
            wait_fill(buf_b, gsem_b)
            writeback(c0 + 1, buf_b, wsem_b)
            wait_drain(buf_a, wsem_a)
            @pl.when(g < n_groups - 1)
            def _():
                gather(c0 + 2, buf_a, gsem_a)
            return carry

        lax.fori_loop(0, n_groups, body, 0)
        wait_drain(buf_b, wsem_b)

    return k(table, idx_flat)


def kernel(token_ids, weight):
    S, T = token_ids.shape
    D = weight.shape[1]
    B = S * T
    idx_flat = token_ids.reshape(B).astype(jnp.int32)
    out = _embedding_gather(weight, idx_flat, B=B, D=D)
    return out.reshape(S, T, D)
